# baseline (speedup 1.0000x reference)
import functools
import math

import jax
import jax.numpy as jnp
from jax import lax
from jax.experimental import pallas as pl
from jax.experimental.pallas import tpu as pltpu
from jax.experimental.pallas import tpu_sc as plsc

D_MODEL = 64
SCALE = math.sqrt(D_MODEL)
LANES = 16
NC, NS = 2, 16
NW = NC * NS
B_ROWS = 4096
SEQ = 200
RW = B_ROWS // NW
IDXC = 128
REM = SEQ - IDXC
RING = 3


def _emb_body(x_hbm, table_hbm, out_hbm, idx_v, b0, b1, b2, g0, g1, g2,
              s0, s1, s2):
    bufs = (b0, b1, b2)
    gsems = (g0, g1, g2)
    ssems = (s0, s1, s2)
    wid = lax.axis_index("s") * NC + lax.axis_index("c")
    base = wid * RW
    pltpu.sync_copy(x_hbm.at[pl.ds(base, RW)], idx_v)

    def fire_gather(b, slot):
        pltpu.async_copy(
            table_hbm.at[idx_v.at[b, pl.ds(0, IDXC)]],
            bufs[slot].at[pl.ds(0, IDXC)],
            gsems[slot],
        )
        pltpu.async_copy(
            table_hbm.at[idx_v.at[b, pl.ds(IDXC, REM)]],
            bufs[slot].at[pl.ds(IDXC, REM)],
            gsems[slot],
        )

    def wait_gather(slot):
        pltpu.make_async_copy(
            table_hbm.at[idx_v.at[0, pl.ds(0, IDXC)]],
            bufs[slot].at[pl.ds(0, IDXC)],
            gsems[slot],
        ).wait()
        pltpu.make_async_copy(
            table_hbm.at[idx_v.at[0, pl.ds(IDXC, REM)]],
            bufs[slot].at[pl.ds(IDXC, REM)],
            gsems[slot],
        ).wait()

    def fire_store(b, slot):
        pltpu.async_copy(bufs[slot], out_hbm.at[base + b], ssems[slot])

    def wait_store(slot):
        pltpu.make_async_copy(bufs[slot], out_hbm.at[base], ssems[slot]).wait()

    def scale(slot):
        buf = bufs[slot]

        @plsc.parallel_loop(0, SEQ, unroll=8)
        def _(r):
            for k in range(D_MODEL // LANES):
                sl = pl.ds(k * LANES, LANES)
                buf[r, sl] = buf[r, sl] * SCALE

    fire_gather(0, 0)
    fire_gather(1, 1)

    def outer(t, carry):
        for p in range(RING):
            g = t * RING + p
            nslot = (p + 2) % RING

            @pl.when(g < RW)
            def _():
                wait_gather(p)
                scale(p)

                @pl.when(g >= 1)
                def _():
                    wait_store(nslot)

                @pl.when(g + 2 < RW)
                def _():
                    fire_gather(g + 2, nslot)

                fire_store(g, p)

        return carry

    lax.fori_loop(0, (RW + RING - 1) // RING, outer, 0)
    wait_store((RW - 1) % RING)


@functools.cache
def _build():
    mesh = plsc.VectorSubcoreMesh(
        core_axis_name="c", subcore_axis_name="s", num_cores=NC, num_subcores=NS
    )
    return functools.partial(
        pl.kernel,
        out_type=jax.ShapeDtypeStruct((B_ROWS, SEQ, D_MODEL), jnp.float32),
        mesh=mesh,
        scratch_types=[
            pltpu.VMEM((RW, SEQ), jnp.int32),
            pltpu.VMEM((SEQ, D_MODEL), jnp.float32),
            pltpu.VMEM((SEQ, D_MODEL), jnp.float32),
            pltpu.VMEM((SEQ, D_MODEL), jnp.float32),
            pltpu.SemaphoreType.DMA,
            pltpu.SemaphoreType.DMA,
            pltpu.SemaphoreType.DMA,
            pltpu.SemaphoreType.DMA,
            pltpu.SemaphoreType.DMA,
            pltpu.SemaphoreType.DMA,
        ],
        compiler_params=pltpu.CompilerParams(use_tc_tiling_on_sc=False),
    )(_emb_body)


def kernel(x, lut_weight):
    return _build()(x.astype(jnp.int32), lut_weight)


# speedup vs baseline: 1.0001x; 1.0001x over previous
import functools
import math

import jax
import jax.numpy as jnp
from jax import lax
from jax.experimental import pallas as pl
from jax.experimental.pallas import tpu as pltpu
from jax.experimental.pallas import tpu_sc as plsc

D_MODEL = 64
SCALE = math.sqrt(D_MODEL)
LANES = 16
NC, NS = 2, 16
NW = NC * NS
B_ROWS = 4096
SEQ = 200
RW = B_ROWS // NW
IDXC = 128
REM = SEQ - IDXC
RING = 3


def _emb_body(x_hbm, table_hbm, out_hbm, idx_v, b0, b1, b2, g0, g1, g2,
              s0, s1, s2):
    bufs = (b0, b1, b2)
    gsems = (g0, g1, g2)
    ssems = (s0, s1, s2)
    wid = lax.axis_index("s") * NC + lax.axis_index("c")
    base = wid * RW
    pltpu.sync_copy(x_hbm.at[pl.ds(base, RW)], idx_v)

    def fire_gather(b, slot):
        pltpu.async_copy(
            table_hbm.at[idx_v.at[b, pl.ds(0, IDXC)]],
            bufs[slot].at[pl.ds(0, IDXC)],
            gsems[slot],
        )
        pltpu.async_copy(
            table_hbm.at[idx_v.at[b, pl.ds(IDXC, REM)]],
            bufs[slot].at[pl.ds(IDXC, REM)],
            gsems[slot],
        )

    def wait_gather(slot):
        pltpu.make_async_copy(
            table_hbm.at[idx_v.at[0, pl.ds(0, IDXC)]],
            bufs[slot].at[pl.ds(0, IDXC)],
            gsems[slot],
        ).wait()
        pltpu.make_async_copy(
            table_hbm.at[idx_v.at[0, pl.ds(IDXC, REM)]],
            bufs[slot].at[pl.ds(IDXC, REM)],
            gsems[slot],
        ).wait()

    def fire_store(b, slot):
        pltpu.async_copy(
            bufs[slot], out_hbm.at[pl.ds((base + b) * SEQ, SEQ)], ssems[slot]
        )

    def wait_store(slot):
        pltpu.make_async_copy(
            bufs[slot], out_hbm.at[pl.ds(base * SEQ, SEQ)], ssems[slot]
        ).wait()

    def scale(slot):
        buf = bufs[slot]

        @plsc.parallel_loop(0, SEQ, unroll=16)
        def _(r):
            for k in range(D_MODEL // LANES):
                sl = pl.ds(k * LANES, LANES)
                buf[r, sl] = buf[r, sl] * SCALE

    fire_gather(0, 0)
    fire_gather(1, 1)

    def outer(t, carry):
        for p in range(RING):
            g = t * RING + p
            nslot = (p + 2) % RING

            @pl.when(g < RW)
            def _():
                wait_gather(p)
                scale(p)

                @pl.when(g >= 1)
                def _():
                    wait_store(nslot)

                @pl.when(g + 2 < RW)
                def _():
                    fire_gather(g + 2, nslot)

                fire_store(g, p)

        return carry

    lax.fori_loop(0, (RW + RING - 1) // RING, outer, 0)
    wait_store((RW - 1) % RING)


@functools.cache
def _build():
    mesh = plsc.VectorSubcoreMesh(
        core_axis_name="c", subcore_axis_name="s", num_cores=NC, num_subcores=NS
    )
    return functools.partial(
        pl.kernel,
        out_type=jax.ShapeDtypeStruct((B_ROWS * SEQ, D_MODEL), jnp.float32),
        mesh=mesh,
        scratch_types=[
            pltpu.VMEM((RW, SEQ), jnp.int32),
            pltpu.VMEM((SEQ, D_MODEL), jnp.float32),
            pltpu.VMEM((SEQ, D_MODEL), jnp.float32),
            pltpu.VMEM((SEQ, D_MODEL), jnp.float32),
            pltpu.SemaphoreType.DMA,
            pltpu.SemaphoreType.DMA,
            pltpu.SemaphoreType.DMA,
            pltpu.SemaphoreType.DMA,
            pltpu.SemaphoreType.DMA,
            pltpu.SemaphoreType.DMA,
        ],
        compiler_params=pltpu.CompilerParams(use_tc_tiling_on_sc=False),
    )(_emb_body)


def kernel(x, lut_weight):
    out = _build()(x.astype(jnp.int32), lut_weight)
    return out.reshape(B_ROWS, SEQ, D_MODEL)


# speedup vs baseline: 1.0014x; 1.0012x over previous
import functools
import math

import jax
import jax.numpy as jnp
from jax import lax
from jax.experimental import pallas as pl
from jax.experimental.pallas import tpu as pltpu
from jax.experimental.pallas import tpu_sc as plsc
from jax.experimental import layout as jex_layout

D_MODEL = 64
SCALE = math.sqrt(D_MODEL)
LANES = 16
NC, NS = 2, 16
NW = NC * NS
B_ROWS = 4096
SEQ = 200
RW = B_ROWS // NW
IDXC = 128
REM = SEQ - IDXC
RING = 3


def _emb_body(x_hbm, table_hbm, out_hbm, idx_v, b0, b1, b2, g0, g1, g2,
              s0, s1, s2):
    bufs = (b0, b1, b2)
    gsems = (g0, g1, g2)
    ssems = (s0, s1, s2)
    wid = lax.axis_index("s") * NC + lax.axis_index("c")
    base = wid * RW
    pltpu.sync_copy(x_hbm.at[pl.ds(base, RW)], idx_v)

    def fire_gather(b, slot):
        pltpu.async_copy(
            table_hbm.at[idx_v.at[b, pl.ds(0, IDXC)]],
            bufs[slot].at[pl.ds(0, IDXC)],
            gsems[slot],
        )
        pltpu.async_copy(
            table_hbm.at[idx_v.at[b, pl.ds(IDXC, REM)]],
            bufs[slot].at[pl.ds(IDXC, REM)],
            gsems[slot],
        )

    def wait_gather(slot):
        pltpu.make_async_copy(
            table_hbm.at[idx_v.at[0, pl.ds(0, IDXC)]],
            bufs[slot].at[pl.ds(0, IDXC)],
            gsems[slot],
        ).wait()
        pltpu.make_async_copy(
            table_hbm.at[idx_v.at[0, pl.ds(IDXC, REM)]],
            bufs[slot].at[pl.ds(IDXC, REM)],
            gsems[slot],
        ).wait()

    def fire_store(b, slot):
        pltpu.async_copy(
            bufs[slot], out_hbm.at[pl.ds((base + b) * SEQ, SEQ)], ssems[slot]
        )

    def wait_store(slot):
        pltpu.make_async_copy(
            bufs[slot], out_hbm.at[pl.ds(base * SEQ, SEQ)], ssems[slot]
        ).wait()

    def scale(slot):
        buf = bufs[slot]

        @plsc.parallel_loop(0, SEQ, unroll=16)
        def _(r):
            for k in range(D_MODEL // LANES):
                sl = pl.ds(k * LANES, LANES)
                buf[r, sl] = buf[r, sl] * SCALE

    fire_gather(0, 0)
    fire_gather(1, 1)

    def outer(t, carry):
        for p in range(RING):
            g = t * RING + p
            nslot = (p + 2) % RING

            @pl.when(g < RW)
            def _():
                wait_gather(p)
                scale(p)

                @pl.when(g >= 1)
                def _():
                    wait_store(nslot)

                @pl.when(g + 2 < RW)
                def _():
                    fire_gather(g + 2, nslot)

                fire_store(g, p)

        return carry

    lax.fori_loop(0, (RW + RING - 1) // RING, outer, 0)
    wait_store((RW - 1) % RING)


@functools.cache
def _build():
    mesh = plsc.VectorSubcoreMesh(
        core_axis_name="c", subcore_axis_name="s", num_cores=NC, num_subcores=NS
    )
    return functools.partial(
        pl.kernel,
        out_type=jax.ShapeDtypeStruct((B_ROWS * SEQ, D_MODEL), jnp.float32),
        mesh=mesh,
        scratch_types=[
            pltpu.VMEM((RW, SEQ), jnp.int32),
            pltpu.VMEM((SEQ, D_MODEL), jnp.float32),
            pltpu.VMEM((SEQ, D_MODEL), jnp.float32),
            pltpu.VMEM((SEQ, D_MODEL), jnp.float32),
            pltpu.SemaphoreType.DMA,
            pltpu.SemaphoreType.DMA,
            pltpu.SemaphoreType.DMA,
            pltpu.SemaphoreType.DMA,
            pltpu.SemaphoreType.DMA,
            pltpu.SemaphoreType.DMA,
        ],
        compiler_params=pltpu.CompilerParams(use_tc_tiling_on_sc=False),
    )(_emb_body)


def kernel(x, lut_weight):
    out = _build()(x.astype(jnp.int32), lut_weight)
    out = jex_layout.with_layout_constraint(
        out,
        jex_layout.Layout(major_to_minor=(0, 1), tiling=((8, 128),)),
    )
    return out.reshape(B_ROWS, SEQ, D_MODEL)
